# Initial kernel scaffold; baseline (speedup 1.0000x reference)
import jax
import jax.numpy as jnp
from jax.experimental import pallas as pl


def kernel(x, emb):
    raise NotImplementedError("write your pallas kernel here")


# speedup vs baseline: 1.6778x; 1.6778x over previous
import jax
import jax.numpy as jnp
from jax.experimental import pallas as pl


_BLK_L = 512


def _add_kernel(x_ref, emb_ref, out_ref):
    out_ref[...] = x_ref[...] + emb_ref[...][None, :, :]


def kernel(x, emb):
    b, seq_len, d = x.shape
    max_len = emb.shape[0]
    if seq_len > max_len:
        x = x[:, :max_len, :]
        seq_len = max_len
    blk_l = _BLK_L if seq_len % _BLK_L == 0 else seq_len
    grid = (seq_len // blk_l, b)
    return pl.pallas_call(
        _add_kernel,
        grid=grid,
        in_specs=[
            pl.BlockSpec((1, blk_l, d), lambda l, bi: (bi, l, 0)),
            pl.BlockSpec((blk_l, d), lambda l, bi: (l, 0)),
        ],
        out_specs=pl.BlockSpec((1, blk_l, d), lambda l, bi: (bi, l, 0)),
        out_shape=jax.ShapeDtypeStruct(x.shape, x.dtype),
    )(x, emb)


# speedup vs baseline: 1.9571x; 1.1665x over previous
import jax
import jax.numpy as jnp
from jax.experimental import pallas as pl


_BLK_L = 512


def _add_kernel(x_ref, emb_ref, out_ref):
    out_ref[...] = x_ref[...] + emb_ref[...][None, :, :]


def kernel(x, emb):
    b, seq_len, d = x.shape
    max_len = emb.shape[0]
    if seq_len > max_len:
        x = x[:, :max_len, :]
        seq_len = max_len
    blk_l = _BLK_L if seq_len % _BLK_L == 0 else seq_len
    grid = (seq_len // blk_l,)
    return pl.pallas_call(
        _add_kernel,
        grid=grid,
        in_specs=[
            pl.BlockSpec((b, blk_l, d), lambda l: (0, l, 0)),
            pl.BlockSpec((blk_l, d), lambda l: (l, 0)),
        ],
        out_specs=pl.BlockSpec((b, blk_l, d), lambda l: (0, l, 0)),
        out_shape=jax.ShapeDtypeStruct(x.shape, x.dtype),
    )(x, emb)
